# baseline (speedup 1.0000x reference)
import functools

import jax
import jax.numpy as jnp
from jax import lax
from jax.experimental import pallas as pl
from jax.experimental.pallas import tpu as pltpu
from jax.experimental.pallas import tpu_sc as plsc

N_U = 8
N_B = 10
R_CUT = 5.0
INV_R_CUT = 1.0 / R_CUT

NUM_CORES = 2
NUM_SUBCORES = 16
LANES = 16
NW = NUM_CORES * NUM_SUBCORES

CHUNK = 2000
ROW = 81
OSTRIDE = 9


def _sc_body(r_hbm, zi_hbm, zj_hbm, w_hbm, out_hbm, wt_v, r_v, zi_v, zj_v,
             out_v, isem0, isem1, osem0, osem1):
    epw = r_hbm.shape[0] // NW
    n_chunks = epw // CHUNK
    cid = lax.axis_index("c")
    sid = lax.axis_index("s")
    wid = sid * NUM_CORES + cid
    base = wid * epw

    in_sems = (isem0, isem1)
    out_sems = (osem0, osem1)

    pltpu.sync_copy(w_hbm, wt_v)

    iota9 = lax.iota(jnp.int32, LANES) * OSTRIDE

    def start_in(b, ci):
        e0 = base + ci * CHUNK
        off = b * CHUNK
        pltpu.async_copy(r_hbm.at[pl.ds(e0, CHUNK)],
                         r_v.at[pl.ds(off, CHUNK)], in_sems[b])
        pltpu.async_copy(zi_hbm.at[pl.ds(e0, CHUNK)],
                         zi_v.at[pl.ds(off, CHUNK)], in_sems[b])
        pltpu.async_copy(zj_hbm.at[pl.ds(e0, CHUNK)],
                         zj_v.at[pl.ds(off, CHUNK)], in_sems[b])

    def wait_in(b):
        off = b * CHUNK
        pltpu.make_async_copy(r_hbm.at[pl.ds(base, CHUNK)],
                              r_v.at[pl.ds(off, CHUNK)], in_sems[b]).wait()
        pltpu.make_async_copy(zi_hbm.at[pl.ds(base, CHUNK)],
                              zi_v.at[pl.ds(off, CHUNK)], in_sems[b]).wait()
        pltpu.make_async_copy(zj_hbm.at[pl.ds(base, CHUNK)],
                              zj_v.at[pl.ds(off, CHUNK)], in_sems[b]).wait()

    def start_out(b, ci):
        e0 = base + ci * CHUNK
        pltpu.async_copy(out_v.at[pl.ds(b * CHUNK * OSTRIDE, CHUNK * OSTRIDE)],
                         out_hbm.at[pl.ds(e0 * OSTRIDE, CHUNK * OSTRIDE)],
                         out_sems[b])

    def wait_out(b):
        pltpu.make_async_copy(
            out_v.at[pl.ds(b * CHUNK * OSTRIDE, CHUNK * OSTRIDE)],
            out_hbm.at[pl.ds(base * OSTRIDE, CHUNK * OSTRIDE)],
            out_sems[b]).wait()

    def compute(b):
        off = b * CHUNK
        ooff = b * CHUNK

        def vec_body(i, c2):
            s = off + i * LANES
            rr = r_v[pl.ds(s, LANES)]
            zi16 = zi_v[pl.ds(s, LANES)]
            zj16 = zj_v[pl.ds(s, LANES)]
            widx = (zi16 * 4 + zj16) * ROW
            t = rr * INV_R_CUT
            env = (1.0 - t) * (1.0 - t)
            env = jnp.where(rr < R_CUT, env, 0.0)
            t2 = t + t
            s_prev = env
            s_cur = t * env
            accs = [None] * N_U
            for u in range(N_U):
                cv = plsc.load_gather(wt_v, [widx + (u * N_B)])
                accs[u] = cv * s_prev
            for u in range(N_U):
                cv = plsc.load_gather(wt_v, [widx + (u * N_B + 1)])
                accs[u] = accs[u] + cv * s_cur
            for bb in range(2, N_B):
                s_next = t2 * s_cur - s_prev
                s_prev = s_cur
                s_cur = s_next
                for u in range(N_U):
                    cv = plsc.load_gather(wt_v, [widx + (u * N_B + bb)])
                    accs[u] = accs[u] + cv * s_cur
            oidx = iota9 + ((ooff + i * LANES) * OSTRIDE)
            for u in range(N_U):
                plsc.store_scatter(out_v, [oidx + u], accs[u])
            return c2

        lax.fori_loop(0, CHUNK // LANES, vec_body, 0)

    start_in(0, 0)

    def chunk_body(ci, carry):
        def process(b):
            @pl.when(ci + 1 < n_chunks)
            def _():
                start_in(1 - b, ci + 1)

            wait_in(b)

            @pl.when(ci >= 2)
            def _():
                wait_out(b)

            compute(b)
            start_out(b, ci)

        @pl.when(ci % 2 == 0)
        def _():
            process(0)

        @pl.when(ci % 2 == 1)
        def _():
            process(1)

        return carry

    lax.fori_loop(0, n_chunks, chunk_body, 0)
    wait_out(0)
    wait_out(1)


def kernel(r, zi, zj, c):
    e = r.shape[0]
    w = c.reshape(16, N_U * N_B)
    w = jnp.pad(w, ((0, 0), (0, ROW - N_U * N_B))).reshape(-1)
    mesh = plsc.VectorSubcoreMesh(core_axis_name="c", subcore_axis_name="s")
    call = functools.partial(
        pl.kernel,
        mesh=mesh,
        compiler_params=pltpu.CompilerParams(needs_layout_passes=False),
        out_type=jax.ShapeDtypeStruct((e * OSTRIDE,), jnp.float32),
        scratch_types=[
            pltpu.VMEM((16 * ROW,), jnp.float32),
            pltpu.VMEM((2 * CHUNK,), jnp.float32),
            pltpu.VMEM((2 * CHUNK,), jnp.int32),
            pltpu.VMEM((2 * CHUNK,), jnp.int32),
            pltpu.VMEM((2 * CHUNK * OSTRIDE,), jnp.float32),
            pltpu.SemaphoreType.DMA,
            pltpu.SemaphoreType.DMA,
            pltpu.SemaphoreType.DMA,
            pltpu.SemaphoreType.DMA,
        ],
    )(_sc_body)
    out9 = call(r, zi, zj, w).reshape(e, OSTRIDE)
    return lax.slice(out9, (0, 0), (e, N_U))


# speedup vs baseline: 5.9851x; 5.9851x over previous
import functools

import jax
import jax.numpy as jnp
from jax import lax
from jax.experimental import pallas as pl
from jax.experimental.pallas import tpu as pltpu
from jax.experimental.pallas import tpu_sc as plsc

N_U = 8
N_B = 10
N_P = N_B // 2
R_CUT = 5.0
INV_R_CUT = 1.0 / R_CUT

NUM_CORES = 2
NUM_SUBCORES = 16
LANES = 16
NW = NUM_CORES * NUM_SUBCORES

TILE = 128
CT = 10

_GDN = jax.lax.GatherDimensionNumbers(
    offset_dims=(), collapsed_slice_dims=(0,), start_index_map=(0,))


def _sc_body(r_hbm, zi_hbm, zj_hbm, w_hbm, z_hbm, out_hbm, wt_v, z_v, r_v,
             zi_v, zj_v, out_v, isem0, isem1, osem0, osem1):
    e = r_hbm.shape[0]
    n_tiles = e // TILE
    tpw = n_tiles // NW
    rem = n_tiles - tpw * NW
    ce = CT * TILE

    cid = lax.axis_index("c")
    sid = lax.axis_index("s")
    wid = sid * NUM_CORES + cid
    t0 = wid * tpw
    n_chunks = (tpw + jnp.where(wid == NW - 1, rem, 0)) // CT
    base = t0 * TILE

    in_sems = (isem0, isem1)
    out_sems = (osem0, osem1)

    pltpu.sync_copy(w_hbm, wt_v)
    pltpu.sync_copy(z_hbm, z_v)
    zvec = z_v[pl.ds(0, LANES)]
    tcols = [wt_v[pl.ds(j * LANES, LANES)] + zvec for j in range(N_U * N_P)]

    def start_in(b, ci):
        e0 = base + ci * ce
        off = b * ce
        pltpu.async_copy(r_hbm.at[pl.ds(e0, ce)],
                         r_v.at[pl.ds(off, ce)], in_sems[b])
        pltpu.async_copy(zi_hbm.at[pl.ds(e0, ce)],
                         zi_v.at[pl.ds(off, ce)], in_sems[b])
        pltpu.async_copy(zj_hbm.at[pl.ds(e0, ce)],
                         zj_v.at[pl.ds(off, ce)], in_sems[b])

    def wait_in(b):
        off = b * ce
        pltpu.make_async_copy(r_hbm.at[pl.ds(base, ce)],
                              r_v.at[pl.ds(off, ce)], in_sems[b]).wait()
        pltpu.make_async_copy(zi_hbm.at[pl.ds(base, ce)],
                              zi_v.at[pl.ds(off, ce)], in_sems[b]).wait()
        pltpu.make_async_copy(zj_hbm.at[pl.ds(base, ce)],
                              zj_v.at[pl.ds(off, ce)], in_sems[b]).wait()

    def start_out(b, ci):
        g0 = t0 + ci * CT
        pltpu.async_copy(out_v.at[pl.ds(b * CT, CT)],
                         out_hbm.at[pl.ds(g0, CT)], out_sems[b])

    def wait_out(b):
        pltpu.make_async_copy(out_v.at[pl.ds(b * CT, CT)],
                              out_hbm.at[pl.ds(t0, CT)], out_sems[b]).wait()

    def compute(b):
        off = b * ce

        def vec_body(i, c2):
            s = off + i * LANES
            rr = r_v[pl.ds(s, LANES)]
            zi16 = zi_v[pl.ds(s, LANES)]
            zj16 = zj_v[pl.ds(s, LANES)]
            jidx_2d = (zi16 * 4 + zj16).reshape(LANES, 1)
            t = rr * INV_R_CUT
            env = (1.0 - t) * (1.0 - t)
            env = jnp.where(rr < R_CUT, env, 0.0)
            t2 = t + t
            accs = [None] * N_U
            s_even = env
            s_odd = t * env
            for p in range(N_P):
                if p > 0:
                    s_even = t2 * s_odd - s_even
                    s_odd = t2 * s_even - s_odd
                for u in range(N_U):
                    pw = lax.gather(tcols[u * N_P + p], jidx_2d, _GDN, (1,),
                                    mode=lax.GatherScatterMode.PROMISE_IN_BOUNDS)
                    chi = plsc.bitcast(pw, jnp.float32)
                    clo = plsc.bitcast(pw << 16, jnp.float32)
                    contrib = chi * s_even + clo * s_odd
                    if p == 0:
                        accs[u] = contrib
                    else:
                        accs[u] = accs[u] + contrib
            g_l = b * CT + (i >> 3)
            el0 = (i & 7) * LANES
            for u in range(N_U):
                out_v[g_l, u, pl.ds(el0, LANES)] = accs[u]
            return c2

        lax.fori_loop(0, ce // LANES, vec_body, 0)

    start_in(0, 0)

    def chunk_body(ci, carry):
        def process(b):
            @pl.when(ci + 1 < n_chunks)
            def _():
                start_in(1 - b, ci + 1)

            wait_in(b)

            @pl.when(ci >= 2)
            def _():
                wait_out(b)

            compute(b)
            start_out(b, ci)

        @pl.when(ci % 2 == 0)
        def _():
            process(0)

        @pl.when(ci % 2 == 1)
        def _():
            process(1)

        return carry

    lax.fori_loop(0, n_chunks, chunk_body, 0)
    wait_out(0)
    wait_out(1)


def kernel(r, zi, zj, c):
    e = r.shape[0]
    cb = lax.bitcast_convert_type(c.astype(jnp.bfloat16), jnp.uint16)
    cb = cb.astype(jnp.uint32).reshape(16, N_U, N_P, 2)
    packed = (cb[..., 0] << 16) | cb[..., 1]
    packed = lax.bitcast_convert_type(packed, jnp.int32)
    w = jnp.transpose(packed, (1, 2, 0)).reshape(-1)
    zeros = jnp.zeros((LANES,), jnp.int32)
    mesh = plsc.VectorSubcoreMesh(core_axis_name="c", subcore_axis_name="s")
    call = functools.partial(
        pl.kernel,
        mesh=mesh,
        compiler_params=pltpu.CompilerParams(needs_layout_passes=False),
        out_type=jax.ShapeDtypeStruct((e // TILE, N_U, TILE), jnp.float32),
        scratch_types=[
            pltpu.VMEM((N_U * N_P * LANES,), jnp.int32),
            pltpu.VMEM((LANES,), jnp.int32),
            pltpu.VMEM((2 * CT * TILE,), jnp.float32),
            pltpu.VMEM((2 * CT * TILE,), jnp.int32),
            pltpu.VMEM((2 * CT * TILE,), jnp.int32),
            pltpu.VMEM((2 * CT, N_U, TILE), jnp.float32),
            pltpu.SemaphoreType.DMA,
            pltpu.SemaphoreType.DMA,
            pltpu.SemaphoreType.DMA,
            pltpu.SemaphoreType.DMA,
        ],
    )(_sc_body)
    out3 = call(r, zi, zj, w, zeros)
    return out3.transpose(0, 2, 1).reshape(e, N_U)


# speedup vs baseline: 6.0459x; 1.0102x over previous
import functools

import jax
import jax.numpy as jnp
from jax import lax
from jax.experimental import pallas as pl
from jax.experimental.pallas import tpu as pltpu
from jax.experimental.pallas import tpu_sc as plsc

N_U = 8
N_B = 10
N_P = N_B // 2
R_CUT = 5.0
INV_R_CUT = 1.0 / R_CUT

NUM_CORES = 2
NUM_SUBCORES = 16
LANES = 16
NW = NUM_CORES * NUM_SUBCORES

TILE = 128
CT = 10

_GDN = jax.lax.GatherDimensionNumbers(
    offset_dims=(), collapsed_slice_dims=(0,), start_index_map=(0,))


def _sc_body(r_hbm, zi_hbm, zj_hbm, w_hbm, z_hbm, out_hbm, wt_v, z_v, r_v,
             zi_v, zj_v, out_v, isem0, isem1, osem0, osem1):
    e = r_hbm.shape[0]
    n_tiles = e // TILE
    ce = CT * TILE

    cid = lax.axis_index("c")
    sid = lax.axis_index("s")
    wid = sid * NUM_CORES + cid
    total_chunks = n_tiles // CT
    cpw = total_chunks // NW
    crem = total_chunks - cpw * NW
    n_chunks = cpw + jnp.where(wid < crem, 1, 0)
    t0 = (wid * cpw + jnp.minimum(wid, crem)) * CT
    base = t0 * TILE

    in_sems = (isem0, isem1)
    out_sems = (osem0, osem1)

    pltpu.sync_copy(w_hbm, wt_v)
    pltpu.sync_copy(z_hbm, z_v)
    zvec = z_v[pl.ds(0, LANES)]
    tcols = [wt_v[pl.ds(j * LANES, LANES)] + zvec for j in range(N_U * N_P)]

    def start_in(b, ci):
        e0 = base + ci * ce
        off = b * ce
        pltpu.async_copy(r_hbm.at[pl.ds(e0, ce)],
                         r_v.at[pl.ds(off, ce)], in_sems[b])
        pltpu.async_copy(zi_hbm.at[pl.ds(e0, ce)],
                         zi_v.at[pl.ds(off, ce)], in_sems[b])
        pltpu.async_copy(zj_hbm.at[pl.ds(e0, ce)],
                         zj_v.at[pl.ds(off, ce)], in_sems[b])

    def wait_in(b):
        off = b * ce
        pltpu.make_async_copy(r_hbm.at[pl.ds(base, ce)],
                              r_v.at[pl.ds(off, ce)], in_sems[b]).wait()
        pltpu.make_async_copy(zi_hbm.at[pl.ds(base, ce)],
                              zi_v.at[pl.ds(off, ce)], in_sems[b]).wait()
        pltpu.make_async_copy(zj_hbm.at[pl.ds(base, ce)],
                              zj_v.at[pl.ds(off, ce)], in_sems[b]).wait()

    def start_out(b, ci):
        g0 = t0 + ci * CT
        pltpu.async_copy(out_v.at[pl.ds(b * CT, CT)],
                         out_hbm.at[pl.ds(g0, CT)], out_sems[b])

    def wait_out(b):
        pltpu.make_async_copy(out_v.at[pl.ds(b * CT, CT)],
                              out_hbm.at[pl.ds(t0, CT)], out_sems[b]).wait()

    def compute(b):
        off = b * ce

        def one_group(i):
            s = off + i * LANES
            rr = r_v[pl.ds(s, LANES)]
            zi16 = zi_v[pl.ds(s, LANES)]
            zj16 = zj_v[pl.ds(s, LANES)]
            jidx_2d = (zi16 * 4 + zj16).reshape(LANES, 1)
            t = rr * INV_R_CUT
            env = (1.0 - t) * (1.0 - t)
            env = jnp.where(rr < R_CUT, env, 0.0)
            t2 = t + t
            accs = [None] * N_U
            s_even = env
            s_odd = t * env
            for p in range(N_P):
                if p > 0:
                    s_even = t2 * s_odd - s_even
                    s_odd = t2 * s_even - s_odd
                for u in range(N_U):
                    pw = lax.gather(tcols[u * N_P + p], jidx_2d, _GDN, (1,),
                                    mode=lax.GatherScatterMode.PROMISE_IN_BOUNDS)
                    chi = plsc.bitcast(pw, jnp.float32)
                    clo = plsc.bitcast(pw << 16, jnp.float32)
                    contrib = chi * s_even + clo * s_odd
                    if p == 0:
                        accs[u] = contrib
                    else:
                        accs[u] = accs[u] + contrib
            g_l = b * CT + (i >> 3)
            el0 = (i & 7) * LANES
            for u in range(N_U):
                out_v[g_l, u, pl.ds(el0, LANES)] = accs[u]

        def vec_body(i2, c2):
            one_group(i2 * 2)
            one_group(i2 * 2 + 1)
            return c2

        lax.fori_loop(0, ce // (2 * LANES), vec_body, 0)

    start_in(0, 0)

    def chunk_body(ci, carry):
        def process(b):
            @pl.when(ci + 1 < n_chunks)
            def _():
                start_in(1 - b, ci + 1)

            wait_in(b)

            @pl.when(ci >= 2)
            def _():
                wait_out(b)

            compute(b)
            start_out(b, ci)

        @pl.when(ci % 2 == 0)
        def _():
            process(0)

        @pl.when(ci % 2 == 1)
        def _():
            process(1)

        return carry

    lax.fori_loop(0, n_chunks, chunk_body, 0)
    wait_out(0)
    wait_out(1)


def kernel(r, zi, zj, c):
    e = r.shape[0]
    cb = lax.bitcast_convert_type(c.astype(jnp.bfloat16), jnp.uint16)
    cb = cb.astype(jnp.uint32).reshape(16, N_U, N_P, 2)
    packed = (cb[..., 0] << 16) | cb[..., 1]
    packed = lax.bitcast_convert_type(packed, jnp.int32)
    w = jnp.transpose(packed, (1, 2, 0)).reshape(-1)
    zeros = jnp.zeros((LANES,), jnp.int32)
    mesh = plsc.VectorSubcoreMesh(core_axis_name="c", subcore_axis_name="s")
    call = functools.partial(
        pl.kernel,
        mesh=mesh,
        compiler_params=pltpu.CompilerParams(needs_layout_passes=False),
        out_type=jax.ShapeDtypeStruct((e // TILE, N_U, TILE), jnp.float32),
        scratch_types=[
            pltpu.VMEM((N_U * N_P * LANES,), jnp.int32),
            pltpu.VMEM((LANES,), jnp.int32),
            pltpu.VMEM((2 * CT * TILE,), jnp.float32),
            pltpu.VMEM((2 * CT * TILE,), jnp.int32),
            pltpu.VMEM((2 * CT * TILE,), jnp.int32),
            pltpu.VMEM((2 * CT, N_U, TILE), jnp.float32),
            pltpu.SemaphoreType.DMA,
            pltpu.SemaphoreType.DMA,
            pltpu.SemaphoreType.DMA,
            pltpu.SemaphoreType.DMA,
        ],
    )(_sc_body)
    out3 = call(r, zi, zj, w, zeros)
    return out3.transpose(0, 2, 1).reshape(e, N_U)
